# Initial kernel scaffold; baseline (speedup 1.0000x reference)
import jax
import jax.numpy as jnp
from jax.experimental import pallas as pl


def kernel(input):
    raise NotImplementedError("write your pallas kernel here")


# speedup vs baseline: 1.6086x; 1.6086x over previous
import jax
import jax.numpy as jnp
from jax.experimental import pallas as pl

_LABELNUM = 1000
_BLK = 512


def _onehot_block(idx_ref, out_ref):
    idx = idx_ref[...]
    cols = jax.lax.broadcasted_iota(jnp.int32, out_ref.shape, 1)
    out_ref[...] = (cols == idx).astype(jnp.float32)


def kernel(input):
    B = input.shape[0]
    idx = input.astype(jnp.int32)
    return pl.pallas_call(
        _onehot_block,
        grid=(B // _BLK,),
        in_specs=[pl.BlockSpec((_BLK, 1), lambda i: (i, 0))],
        out_specs=pl.BlockSpec((_BLK, _LABELNUM), lambda i: (i, 0)),
        out_shape=jax.ShapeDtypeStruct((B, _LABELNUM), jnp.float32),
    )(idx)


# speedup vs baseline: 1.7971x; 1.1171x over previous
import jax
import jax.numpy as jnp
from jax.experimental import pallas as pl

_LABELNUM = 1000
_BLK = 2048


def _onehot_block(idx_ref, out_ref):
    idx = idx_ref[...]
    cols = jax.lax.broadcasted_iota(jnp.int32, out_ref.shape, 1)
    out_ref[...] = (cols == idx).astype(jnp.float32)


def kernel(input):
    B = input.shape[0]
    idx = input.astype(jnp.int32)
    return pl.pallas_call(
        _onehot_block,
        grid=(B // _BLK,),
        in_specs=[pl.BlockSpec((_BLK, 1), lambda i: (i, 0))],
        out_specs=pl.BlockSpec((_BLK, _LABELNUM), lambda i: (i, 0)),
        out_shape=jax.ShapeDtypeStruct((B, _LABELNUM), jnp.float32),
    )(idx)


# speedup vs baseline: 1.7982x; 1.0007x over previous
import jax
import jax.numpy as jnp
from jax.experimental import pallas as pl

_LABELNUM = 1000
_BLK = 4096


def _onehot_block(idx_ref, out_ref):
    idx = idx_ref[...]
    cols = jax.lax.broadcasted_iota(jnp.int32, out_ref.shape, 1)
    out_ref[...] = (cols == idx).astype(jnp.float32)


def kernel(input):
    B = input.shape[0]
    idx = input.astype(jnp.int32)
    return pl.pallas_call(
        _onehot_block,
        grid=(B // _BLK,),
        in_specs=[pl.BlockSpec((_BLK, 1), lambda i: (i, 0))],
        out_specs=pl.BlockSpec((_BLK, _LABELNUM), lambda i: (i, 0)),
        out_shape=jax.ShapeDtypeStruct((B, _LABELNUM), jnp.float32),
    )(idx)
